# Initial kernel scaffold; baseline (speedup 1.0000x reference)
import jax
import jax.numpy as jnp
from jax.experimental import pallas as pl


def kernel(x, edge_index, W1, b1, W2, b2):
    raise NotImplementedError("write your pallas kernel here")


# speedup vs baseline: 46.1199x; 46.1199x over previous
import functools

import jax
import jax.numpy as jnp
from jax import lax
from jax.experimental import pallas as pl
from jax.experimental.pallas import tpu as pltpu
from jax.experimental.pallas import tpu_sc as plsc

_NC, _NS, _L = 2, 16, 16
_NW = _NC * _NS
_CH = 128
_CG = 16


def _make_deg_kernel(NP, EPW, NW):
    mesh = plsc.VectorSubcoreMesh(core_axis_name="c", subcore_axis_name="s",
                                  num_cores=_NC, num_subcores=_NS)

    @functools.partial(
        pl.kernel, mesh=mesh,
        out_type=jax.ShapeDtypeStruct((NW, NP), jnp.float32),
        scratch_types=[
            pltpu.VMEM((EPW,), jnp.int32),
            pltpu.VMEM((NP,), jnp.float32),
        ],
        compiler_params=pltpu.CompilerParams(needs_layout_passes=False),
    )
    def deg_kernel(dst_hbm, zeros_hbm, out_hbm, idx_v, hist_v):
        wid = lax.axis_index("s") * _NC + lax.axis_index("c")
        pltpu.sync_copy(dst_hbm.at[pl.ds(wid * EPW, EPW)], idx_v)
        pltpu.sync_copy(zeros_hbm, hist_v)
        ones = jnp.full((_L,), 1.0, jnp.float32)

        def body(i, carry):
            idx = idx_v[pl.ds(i * _L, _L)]
            plsc.addupdate_scatter(hist_v, [idx], ones)
            return carry

        lax.fori_loop(0, EPW // _L, body, 0)
        pltpu.sync_copy(hist_v, out_hbm.at[wid])

    return deg_kernel


def _make_agg_kernel(NP, NCH, D):
    mesh = plsc.VectorSubcoreMesh(core_axis_name="c", subcore_axis_name="s",
                                  num_cores=_NC, num_subcores=_NS)
    RPT = NP // _NS

    @functools.partial(
        pl.kernel, mesh=mesh,
        out_type=jax.ShapeDtypeStruct((_NC, NP, D), jnp.float32),
        scratch_types=[
            pltpu.VMEM((NCH, _CH), jnp.int32),
            pltpu.VMEM((NCH, _CH), jnp.int32),
            pltpu.VMEM((_CG * _CH, D), jnp.float32),
            pltpu.VMEM_SHARED((NP, D), jnp.float32),
            pltpu.SemaphoreType.DMA,
            pltpu.SemaphoreType.DMA,
        ],
        compiler_params=pltpu.CompilerParams(
            needs_layout_passes=False, use_tc_tiling_on_sc=False),
    )
    def agg_kernel(y_hbm, src_hbm, dst_hbm, zrows_hbm, out_hbm,
                   sidx_v, didx_v, rows_v, acc_sh, gsem, ssem):
        c = lax.axis_index("c")
        s = lax.axis_index("s")
        wid = s * _NC + c

        pltpu.sync_copy(zrows_hbm.at[pl.ds(s * RPT, RPT)],
                        acc_sh.at[pl.ds(s * RPT, RPT)])
        pltpu.sync_copy(src_hbm.at[wid], sidx_v)
        pltpu.sync_copy(dst_hbm.at[wid], didx_v)
        plsc.subcore_barrier()

        for g0 in range(0, NCH, _CG):
            n = min(_CG, NCH - g0)
            descs = [
                pltpu.async_copy(y_hbm.at[sidx_v.at[g0 + j]],
                                 rows_v.at[pl.ds(j * _CH, _CH)], gsem)
                for j in range(n)
            ]
            for d in descs:
                d.wait()
            descs = [
                pltpu.async_copy(rows_v.at[pl.ds(j * _CH, _CH)],
                                 acc_sh.at[didx_v.at[g0 + j]], ssem, add=True)
                for j in range(n)
            ]
            for d in descs:
                d.wait()

        plsc.subcore_barrier()
        pltpu.sync_copy(acc_sh.at[pl.ds(s * RPT, RPT)],
                        out_hbm.at[c, pl.ds(s * RPT, RPT), :])

    return agg_kernel


def _tc1_body(parts_ref, x_ref, w1_ref, dis_ref, y1_ref):
    N = x_ref.shape[0]
    D = w1_ref.shape[1]
    deg = jnp.sum(parts_ref[...], axis=0)[:N] + 1.0
    dis = lax.rsqrt(deg)
    dis16 = jnp.broadcast_to(dis[:, None], (N, D))
    dis_ref[...] = dis16
    xw = jnp.dot(x_ref[...], w1_ref[...], preferred_element_type=jnp.float32)
    y1_ref[...] = dis16 * xw


def _tc2_body(agg_ref, y1_ref, dis_ref, b1_ref, y2_ref):
    a = agg_ref[0] + agg_ref[1] + y1_ref[...]
    h = jnp.maximum(dis_ref[...] * a + b1_ref[...], 0.0)
    y2_ref[...] = dis_ref[...] * h


def _tc3_body(agg_ref, y2_ref, dis_ref, w2_ref, b2_ref, out_ref):
    a2 = dis_ref[...] * (agg_ref[0] + agg_ref[1] + y2_ref[...])
    out_ref[...] = (
        jnp.dot(a2, w2_ref[...], preferred_element_type=jnp.float32)
        + b2_ref[...]
    )


def kernel(x, edge_index, W1, b1, W2, b2):
    N, D_in = x.shape
    D_hid = W1.shape[1]
    D_out = W2.shape[1]
    E = edge_index.shape[1]

    NP = -(-(N + 1) // 128) * 128
    NCH = -(-E // (_NW * _CH))
    EPW = NCH * _CH
    E_pad = _NW * EPW

    src = edge_index[0].astype(jnp.int32)
    dst = edge_index[1].astype(jnp.int32)
    if E_pad > E:
        fill = jnp.full((E_pad - E,), N, jnp.int32)
        src = jnp.concatenate([src, fill])
        dst = jnp.concatenate([dst, fill])
    srcR = src.reshape(_NW, NCH, _CH)
    dstR = dst.reshape(_NW, NCH, _CH)

    zeros_np = jnp.zeros((NP,), jnp.float32)
    zrows = jnp.zeros((NP, D_hid), jnp.float32)
    pad_rows = jnp.zeros((NP - N, D_hid), jnp.float32)

    deg_parts = _make_deg_kernel(NP, EPW, _NW)(dst, zeros_np)

    dis16, y1 = pl.pallas_call(
        _tc1_body,
        out_shape=[
            jax.ShapeDtypeStruct((N, D_hid), jnp.float32),
            jax.ShapeDtypeStruct((N, D_hid), jnp.float32),
        ],
    )(deg_parts, x, W1)

    agg_kernel = _make_agg_kernel(NP, NCH, D_hid)

    agg1 = agg_kernel(jnp.concatenate([y1, pad_rows]), srcR, dstR, zrows)

    y2 = pl.pallas_call(
        _tc2_body,
        out_shape=jax.ShapeDtypeStruct((N, D_hid), jnp.float32),
    )(agg1[:, :N, :], y1, dis16, b1.reshape(1, D_hid))

    agg2 = agg_kernel(jnp.concatenate([y2, pad_rows]), srcR, dstR, zrows)

    out = pl.pallas_call(
        _tc3_body,
        out_shape=jax.ShapeDtypeStruct((N, D_out), jnp.float32),
    )(agg2[:, :N, :], y2, dis16, W2, b2.reshape(1, D_out))

    return out


# speedup vs baseline: 48.7685x; 1.0574x over previous
import functools

import jax
import jax.numpy as jnp
from jax import lax
from jax.experimental import pallas as pl
from jax.experimental.pallas import tpu as pltpu
from jax.experimental.pallas import tpu_sc as plsc

_NC, _NS, _L = 2, 16, 16
_NW = _NC * _NS
_CH = 128
_CG = 16


def _make_deg_kernel(NP, EPW, NW):
    mesh = plsc.VectorSubcoreMesh(core_axis_name="c", subcore_axis_name="s",
                                  num_cores=_NC, num_subcores=_NS)

    @functools.partial(
        pl.kernel, mesh=mesh,
        out_type=jax.ShapeDtypeStruct((NW, NP), jnp.float32),
        scratch_types=[
            pltpu.VMEM((EPW,), jnp.int32),
            pltpu.VMEM((NP,), jnp.float32),
        ],
        compiler_params=pltpu.CompilerParams(needs_layout_passes=False),
    )
    def deg_kernel(dst_hbm, zeros_hbm, out_hbm, idx_v, hist_v):
        wid = lax.axis_index("s") * _NC + lax.axis_index("c")
        pltpu.sync_copy(dst_hbm.at[pl.ds(wid * EPW, EPW)], idx_v)
        pltpu.sync_copy(zeros_hbm, hist_v)
        ones = jnp.full((_L,), 1.0, jnp.float32)

        def body(i, carry):
            for u in range(4):
                idx = idx_v[pl.ds((i * 4 + u) * _L, _L)]
                plsc.addupdate_scatter(hist_v, [idx], ones)
            return carry

        lax.fori_loop(0, EPW // (4 * _L), body, 0)
        pltpu.sync_copy(hist_v, out_hbm.at[wid])

    return deg_kernel


def _make_agg_kernel(NP, NCH, D):
    mesh = plsc.VectorSubcoreMesh(core_axis_name="c", subcore_axis_name="s",
                                  num_cores=_NC, num_subcores=_NS)
    RPT = NP // _NS

    NG = -(-NCH // _CG)

    @functools.partial(
        pl.kernel, mesh=mesh,
        out_type=jax.ShapeDtypeStruct((_NC, NP, D), jnp.float32),
        scratch_types=[
            pltpu.VMEM((NCH, _CH), jnp.int32),
            pltpu.VMEM((NCH, _CH), jnp.int32),
            pltpu.VMEM((2 * _CG * _CH, D), jnp.float32),
            pltpu.VMEM_SHARED((NP, D), jnp.float32),
            pltpu.SemaphoreType.DMA,
            pltpu.SemaphoreType.DMA,
            pltpu.SemaphoreType.DMA,
            pltpu.SemaphoreType.DMA,
        ],
        compiler_params=pltpu.CompilerParams(
            needs_layout_passes=False, use_tc_tiling_on_sc=False),
    )
    def agg_kernel(y_hbm, src_hbm, dst_hbm, zrows_hbm, out_hbm,
                   sidx_v, didx_v, rows_v, acc_sh,
                   gsem0, gsem1, ssem0, ssem1):
        c = lax.axis_index("c")
        s = lax.axis_index("s")
        wid = s * _NC + c
        gsems = (gsem0, gsem1)
        ssems = (ssem0, ssem1)

        pltpu.sync_copy(zrows_hbm.at[pl.ds(s * RPT, RPT)],
                        acc_sh.at[pl.ds(s * RPT, RPT)])
        pltpu.sync_copy(src_hbm.at[wid], sidx_v)
        pltpu.sync_copy(dst_hbm.at[wid], didx_v)
        plsc.subcore_barrier()

        def chunks(g):
            return range(g * _CG, min((g + 1) * _CG, NCH))

        def issue_gathers(g):
            buf = (g % 2) * _CG * _CH
            return [
                pltpu.async_copy(
                    y_hbm.at[sidx_v.at[k]],
                    rows_v.at[pl.ds(buf + (k - g * _CG) * _CH, _CH)],
                    gsems[g % 2])
                for k in chunks(g)
            ]

        def issue_scatters(g):
            buf = (g % 2) * _CG * _CH
            return [
                pltpu.async_copy(
                    rows_v.at[pl.ds(buf + (k - g * _CG) * _CH, _CH)],
                    acc_sh.at[didx_v.at[k]],
                    ssems[g % 2], add=True)
                for k in chunks(g)
            ]

        gd = {0: issue_gathers(0)}
        sd = {}
        for g in range(NG):
            if g + 1 < NG:
                for d in sd.pop(g - 1, ()):
                    d.wait()
                gd[g + 1] = issue_gathers(g + 1)
            for d in gd.pop(g):
                d.wait()
            sd[g] = issue_scatters(g)
        for g in sorted(sd):
            for d in sd[g]:
                d.wait()

        plsc.subcore_barrier()
        pltpu.sync_copy(acc_sh.at[pl.ds(s * RPT, RPT)],
                        out_hbm.at[c, pl.ds(s * RPT, RPT), :])

    return agg_kernel


def _tc1_body(parts_ref, x_ref, w1_ref, dis_ref, y1_ref):
    N = x_ref.shape[0]
    D = w1_ref.shape[1]
    deg = jnp.sum(parts_ref[...], axis=0)[:N] + 1.0
    dis = lax.rsqrt(deg)
    dis16 = jnp.broadcast_to(dis[:, None], (N, D))
    dis_ref[...] = dis16
    xw = jnp.dot(x_ref[...], w1_ref[...], preferred_element_type=jnp.float32)
    y1_ref[...] = dis16 * xw


def _tc2_body(agg_ref, y1_ref, dis_ref, b1_ref, y2_ref):
    a = agg_ref[0] + agg_ref[1] + y1_ref[...]
    h = jnp.maximum(dis_ref[...] * a + b1_ref[...], 0.0)
    y2_ref[...] = dis_ref[...] * h


def _tc3_body(agg_ref, y2_ref, dis_ref, w2_ref, b2_ref, out_ref):
    a2 = dis_ref[...] * (agg_ref[0] + agg_ref[1] + y2_ref[...])
    out_ref[...] = (
        jnp.dot(a2, w2_ref[...], preferred_element_type=jnp.float32)
        + b2_ref[...]
    )


def kernel(x, edge_index, W1, b1, W2, b2):
    N, D_in = x.shape
    D_hid = W1.shape[1]
    D_out = W2.shape[1]
    E = edge_index.shape[1]

    NP = -(-(N + 1) // 128) * 128
    NCH = -(-E // (_NW * _CH))
    EPW = NCH * _CH
    E_pad = _NW * EPW

    src = edge_index[0].astype(jnp.int32)
    dst = edge_index[1].astype(jnp.int32)
    if E_pad > E:
        fill = jnp.full((E_pad - E,), N, jnp.int32)
        src = jnp.concatenate([src, fill])
        dst = jnp.concatenate([dst, fill])
    srcR = src.reshape(_NW, NCH, _CH)
    dstR = dst.reshape(_NW, NCH, _CH)

    zeros_np = jnp.zeros((NP,), jnp.float32)
    zrows = jnp.zeros((NP, D_hid), jnp.float32)
    pad_rows = jnp.zeros((NP - N, D_hid), jnp.float32)

    deg_parts = _make_deg_kernel(NP, EPW, _NW)(dst, zeros_np)

    dis16, y1 = pl.pallas_call(
        _tc1_body,
        out_shape=[
            jax.ShapeDtypeStruct((N, D_hid), jnp.float32),
            jax.ShapeDtypeStruct((N, D_hid), jnp.float32),
        ],
    )(deg_parts, x, W1)

    agg_kernel = _make_agg_kernel(NP, NCH, D_hid)

    agg1 = agg_kernel(jnp.concatenate([y1, pad_rows]), srcR, dstR, zrows)

    y2 = pl.pallas_call(
        _tc2_body,
        out_shape=jax.ShapeDtypeStruct((N, D_hid), jnp.float32),
    )(agg1[:, :N, :], y1, dis16, b1.reshape(1, D_hid))

    agg2 = agg_kernel(jnp.concatenate([y2, pad_rows]), srcR, dstR, zrows)

    out = pl.pallas_call(
        _tc3_body,
        out_shape=jax.ShapeDtypeStruct((N, D_out), jnp.float32),
    )(agg2[:, :N, :], y2, dis16, W2, b2.reshape(1, D_out))

    return out


# speedup vs baseline: 59.1882x; 1.2137x over previous
import functools

import jax
import jax.numpy as jnp
from jax import lax
from jax.experimental import pallas as pl
from jax.experimental.pallas import tpu as pltpu
from jax.experimental.pallas import tpu_sc as plsc

_NC, _NS, _L = 2, 16, 16
_NW = _NC * _NS
_CH = 128
_CG = 16


def _make_deg_kernel(NP, NCH, NW):
    mesh = plsc.VectorSubcoreMesh(core_axis_name="c", subcore_axis_name="s",
                                  num_cores=_NC, num_subcores=_NS)

    @functools.partial(
        pl.kernel, mesh=mesh,
        out_type=jax.ShapeDtypeStruct((NW, NP), jnp.float32),
        scratch_types=[
            pltpu.VMEM((NCH, _CH), jnp.int32),
            pltpu.VMEM((NP,), jnp.float32),
        ],
        compiler_params=pltpu.CompilerParams(needs_layout_passes=False),
    )
    def deg_kernel(ei_hbm, zeros_hbm, out_hbm, idx_v, hist_v):
        wid = lax.axis_index("s") * _NC + lax.axis_index("c")
        pltpu.sync_copy(ei_hbm.at[1, wid], idx_v)
        pltpu.sync_copy(zeros_hbm, hist_v)
        ones = jnp.full((_L,), 1.0, jnp.float32)

        def body(i, carry):
            for u in range(_CH // _L):
                idx = idx_v[i, pl.ds(u * _L, _L)]
                plsc.addupdate_scatter(hist_v, [idx], ones)
            return carry

        lax.fori_loop(0, NCH, body, 0)
        pltpu.sync_copy(hist_v, out_hbm.at[wid])

    return deg_kernel


def _make_agg_kernel(NP, NCH, D):
    mesh = plsc.VectorSubcoreMesh(core_axis_name="c", subcore_axis_name="s",
                                  num_cores=_NC, num_subcores=_NS)
    RPT = NP // _NS

    NG = -(-NCH // _CG)

    @functools.partial(
        pl.kernel, mesh=mesh,
        out_type=jax.ShapeDtypeStruct((_NC, NP, D), jnp.float32),
        scratch_types=[
            pltpu.VMEM((NCH, _CH), jnp.int32),
            pltpu.VMEM((NCH, _CH), jnp.int32),
            pltpu.VMEM((2 * _CG * _CH, D), jnp.float32),
            pltpu.VMEM_SHARED((NP, D), jnp.float32),
            pltpu.SemaphoreType.DMA,
            pltpu.SemaphoreType.DMA,
            pltpu.SemaphoreType.DMA,
            pltpu.SemaphoreType.DMA,
        ],
        compiler_params=pltpu.CompilerParams(
            needs_layout_passes=False, use_tc_tiling_on_sc=False),
    )
    def agg_kernel(y_hbm, ei_hbm, zrows_hbm, out_hbm,
                   sidx_v, didx_v, rows_v, acc_sh,
                   gsem0, gsem1, ssem0, ssem1):
        c = lax.axis_index("c")
        s = lax.axis_index("s")
        wid = s * _NC + c
        gsems = (gsem0, gsem1)
        ssems = (ssem0, ssem1)

        pltpu.sync_copy(zrows_hbm.at[pl.ds(s * RPT, RPT)],
                        acc_sh.at[pl.ds(s * RPT, RPT)])
        pltpu.sync_copy(ei_hbm.at[0, wid], sidx_v)
        pltpu.sync_copy(ei_hbm.at[1, wid], didx_v)
        plsc.subcore_barrier()

        def chunks(g):
            return range(g * _CG, min((g + 1) * _CG, NCH))

        def issue_gathers(g):
            buf = (g % 2) * _CG * _CH
            return [
                pltpu.async_copy(
                    y_hbm.at[sidx_v.at[k]],
                    rows_v.at[pl.ds(buf + (k - g * _CG) * _CH, _CH)],
                    gsems[g % 2])
                for k in chunks(g)
            ]

        def issue_scatters(g):
            buf = (g % 2) * _CG * _CH
            return [
                pltpu.async_copy(
                    rows_v.at[pl.ds(buf + (k - g * _CG) * _CH, _CH)],
                    acc_sh.at[didx_v.at[k]],
                    ssems[g % 2], add=True)
                for k in chunks(g)
            ]

        gd = {0: issue_gathers(0)}
        sd = {}
        for g in range(NG):
            if g + 1 < NG:
                for d in sd.pop(g - 1, ()):
                    d.wait()
                gd[g + 1] = issue_gathers(g + 1)
            for d in gd.pop(g):
                d.wait()
            sd[g] = issue_scatters(g)
        for g in sorted(sd):
            for d in sd[g]:
                d.wait()

        plsc.subcore_barrier()
        pltpu.sync_copy(acc_sh.at[pl.ds(s * RPT, RPT)],
                        out_hbm.at[c, pl.ds(s * RPT, RPT), :])

    return agg_kernel


def _tc_mm_body(x_ref, w1_ref, xw_ref):
    xw_ref[...] = jnp.dot(x_ref[...], w1_ref[...],
                          preferred_element_type=jnp.float32)


def _tc1_body(parts_ref, xw_ref, dis_ref, y1_ref):
    N = xw_ref.shape[0]
    NP = dis_ref.shape[0]
    D = xw_ref.shape[1]
    deg = jnp.sum(parts_ref[...], axis=0) + 1.0
    dis = lax.rsqrt(deg)
    dis_ref[...] = jnp.broadcast_to(dis[:, None], (NP, D))
    y1_ref[...] = dis_ref[...] * jnp.pad(xw_ref[...], ((0, NP - N), (0, 0)))


def _tc2_body(agg_ref, y1_ref, dis_ref, b1_ref, y2_ref):
    a = agg_ref[0] + agg_ref[1] + y1_ref[...]
    h = jnp.maximum(dis_ref[...] * a + b1_ref[...], 0.0)
    y2_ref[...] = dis_ref[...] * h


def _tc3_body(agg_ref, y2_ref, dis_ref, w2_ref, b2_ref, out_ref):
    N = out_ref.shape[0]
    a2 = dis_ref[...] * (agg_ref[0] + agg_ref[1] + y2_ref[...])
    out_ref[...] = (
        jnp.dot(a2[:N, :], w2_ref[...], preferred_element_type=jnp.float32)
        + b2_ref[...]
    )


def kernel(x, edge_index, W1, b1, W2, b2):
    N, D_in = x.shape
    D_hid = W1.shape[1]
    D_out = W2.shape[1]
    E = edge_index.shape[1]

    NP = -(-(N + 1) // 128) * 128
    NCH = -(-E // (_NW * _CH))
    EPW = NCH * _CH
    E_pad = _NW * EPW

    ei = edge_index.astype(jnp.int32)
    ei = jnp.pad(ei, ((0, 0), (0, E_pad - E)), constant_values=N)
    ei = lax.optimization_barrier(ei.reshape(2, _NW, NCH, _CH))

    zeros_np = jnp.zeros((NP,), jnp.float32)
    zrows = jnp.zeros((NP, D_hid), jnp.float32)

    deg_parts = _make_deg_kernel(NP, NCH, _NW)(ei, zeros_np)

    xw = pl.pallas_call(
        _tc_mm_body,
        out_shape=jax.ShapeDtypeStruct((N, D_hid), jnp.float32),
    )(x, W1)

    dis16, y1 = pl.pallas_call(
        _tc1_body,
        out_shape=[
            jax.ShapeDtypeStruct((NP, D_hid), jnp.float32),
            jax.ShapeDtypeStruct((NP, D_hid), jnp.float32),
        ],
    )(deg_parts, xw)

    agg_kernel = _make_agg_kernel(NP, NCH, D_hid)

    agg1 = agg_kernel(y1, ei, zrows)

    y2 = pl.pallas_call(
        _tc2_body,
        out_shape=jax.ShapeDtypeStruct((NP, D_hid), jnp.float32),
    )(agg1, y1, dis16, b1.reshape(1, D_hid))

    agg2 = agg_kernel(y2, ei, zrows)

    out = pl.pallas_call(
        _tc3_body,
        out_shape=jax.ShapeDtypeStruct((N, D_out), jnp.float32),
    )(agg2, y2, dis16, W2, b2.reshape(1, D_out))

    return out


# speedup vs baseline: 60.0386x; 1.0144x over previous
import functools

import jax
import jax.numpy as jnp
from jax import lax
from jax.experimental import pallas as pl
from jax.experimental.pallas import tpu as pltpu
from jax.experimental.pallas import tpu_sc as plsc

_NC, _NS, _L = 2, 16, 16
_NW = _NC * _NS
_CH = 128
_CG = 16


def _make_deg_kernel(NP, NCH, NW):
    mesh = plsc.VectorSubcoreMesh(core_axis_name="c", subcore_axis_name="s",
                                  num_cores=_NC, num_subcores=_NS)

    @functools.partial(
        pl.kernel, mesh=mesh,
        out_type=jax.ShapeDtypeStruct((NW, NP), jnp.float32),
        scratch_types=[
            pltpu.VMEM((NCH, _CH), jnp.int32),
            pltpu.VMEM((NP,), jnp.float32),
        ],
        compiler_params=pltpu.CompilerParams(needs_layout_passes=False),
    )
    def deg_kernel(ei_hbm, zeros_hbm, out_hbm, idx_v, hist_v):
        wid = lax.axis_index("s") * _NC + lax.axis_index("c")
        pltpu.sync_copy(ei_hbm.at[1, wid], idx_v)
        pltpu.sync_copy(zeros_hbm, hist_v)
        ones = jnp.full((_L,), 1.0, jnp.float32)

        def body(i, carry):
            for u in range(_CH // _L):
                idx = idx_v[i, pl.ds(u * _L, _L)]
                plsc.addupdate_scatter(hist_v, [idx], ones)
            return carry

        lax.fori_loop(0, NCH, body, 0)
        pltpu.sync_copy(hist_v, out_hbm.at[wid])

    return deg_kernel


def _make_agg_kernel(NP, NCH, D):
    mesh = plsc.VectorSubcoreMesh(core_axis_name="c", subcore_axis_name="s",
                                  num_cores=_NC, num_subcores=_NS)
    RPT = NP // _NS
    FPT = RPT * D // 128

    NG = -(-NCH // _CG)

    @functools.partial(
        pl.kernel, mesh=mesh,
        out_type=jax.ShapeDtypeStruct((_NC, NP, D), jnp.float32),
        scratch_types=[
            pltpu.VMEM((NCH, _CH), jnp.int32),
            pltpu.VMEM((NCH, _CH), jnp.int32),
            pltpu.VMEM((2 * _CG * _CH, D), jnp.float32),
            pltpu.VMEM_SHARED((NP, D), jnp.float32),
            pltpu.SemaphoreType.DMA,
            pltpu.SemaphoreType.DMA,
            pltpu.SemaphoreType.DMA,
            pltpu.SemaphoreType.DMA,
        ],
        compiler_params=pltpu.CompilerParams(
            needs_layout_passes=False, use_tc_tiling_on_sc=False),
    )
    def agg_kernel(y_hbm, ei_hbm, zrows_hbm, out_hbm,
                   sidx_v, didx_v, rows_v, acc_sh,
                   gsem0, gsem1, ssem0, ssem1):
        c = lax.axis_index("c")
        s = lax.axis_index("s")
        wid = s * _NC + c
        gsems = (gsem0, gsem1)
        ssems = (ssem0, ssem1)
        with jax.named_scope("agg_stage"):
            pltpu.sync_copy(zrows_hbm.at[pl.ds(s * RPT, RPT)],
                            acc_sh.at[pl.ds(s * RPT, RPT)])
            pltpu.sync_copy(ei_hbm.at[0, wid], sidx_v)
            pltpu.sync_copy(ei_hbm.at[1, wid], didx_v)
            plsc.subcore_barrier()

        def chunks(g):
            return range(g * _CG, min((g + 1) * _CG, NCH))

        def issue_gathers(g):
            buf = (g % 2) * _CG * _CH
            return [
                pltpu.async_copy(
                    y_hbm.at[sidx_v.at[k]],
                    rows_v.at[pl.ds(buf + (k - g * _CG) * _CH, _CH)],
                    gsems[g % 2])
                for k in chunks(g)
            ]

        def issue_scatters(g):
            buf = (g % 2) * _CG * _CH
            return [
                pltpu.async_copy(
                    rows_v.at[pl.ds(buf + (k - g * _CG) * _CH, _CH)],
                    acc_sh.at[didx_v.at[k]],
                    ssems[g % 2], add=True)
                for k in chunks(g)
            ]

        with jax.named_scope("agg_edges"):
            gd = {0: issue_gathers(0)}
            sd = {}
            for g in range(NG):
                if g + 1 < NG:
                    for d in sd.pop(g - 1, ()):
                        d.wait()
                    gd[g + 1] = issue_gathers(g + 1)
                for d in gd.pop(g):
                    d.wait()
                sd[g] = issue_scatters(g)
            for g in sorted(sd):
                for d in sd[g]:
                    d.wait()

        with jax.named_scope("agg_dump"):
            plsc.subcore_barrier()
            pltpu.sync_copy(acc_sh.at[pl.ds(s * RPT, RPT)],
                            out_hbm.at[c, pl.ds(s * RPT, RPT), :])

    return agg_kernel


def _tc_mm_body(x_ref, w1_ref, xw_ref):
    xw_ref[...] = jnp.dot(x_ref[...], w1_ref[...],
                          preferred_element_type=jnp.float32)


def _tc1_body(parts_ref, xw_ref, dis_ref, y1_ref):
    N = xw_ref.shape[0]
    NP = dis_ref.shape[0]
    D = xw_ref.shape[1]
    deg = jnp.sum(parts_ref[...], axis=0) + 1.0
    dis = lax.rsqrt(deg)
    dis_ref[...] = jnp.broadcast_to(dis[:, None], (NP, D))
    y1_ref[...] = dis_ref[...] * jnp.pad(xw_ref[...], ((0, NP - N), (0, 0)))


def _tc2_body(agg_ref, y1_ref, dis_ref, b1_ref, y2_ref):
    a = agg_ref[0] + agg_ref[1] + y1_ref[...]
    h = jnp.maximum(dis_ref[...] * a + b1_ref[...], 0.0)
    y2_ref[...] = dis_ref[...] * h


def _tc3_body(agg_ref, y2_ref, dis_ref, w2_ref, b2_ref, out_ref):
    N = out_ref.shape[0]
    a2 = dis_ref[...] * (agg_ref[0] + agg_ref[1] + y2_ref[...])
    out_ref[...] = (
        jnp.dot(a2[:N, :], w2_ref[...], preferred_element_type=jnp.float32)
        + b2_ref[...]
    )


def kernel(x, edge_index, W1, b1, W2, b2):
    N, D_in = x.shape
    D_hid = W1.shape[1]
    D_out = W2.shape[1]
    E = edge_index.shape[1]

    NP = -(-(N + 1) // 128) * 128
    NCH = -(-E // (_NW * _CH))
    EPW = NCH * _CH
    E_pad = _NW * EPW
    F = NP * D_hid // 128
    NF = N * D_hid // 128
    FPT = F // _NS

    ei = edge_index.astype(jnp.int32)
    ei = jnp.pad(ei, ((0, 0), (0, E_pad - E)), constant_values=N)
    ei = lax.optimization_barrier(ei.reshape(2, _NW, NCH, _CH))

    zeros_np = jnp.zeros((NP,), jnp.float32)
    zrows = jnp.zeros((NP, D_hid), jnp.float32)

    deg_parts = _make_deg_kernel(NP, NCH, _NW)(ei, zeros_np)

    xw = pl.pallas_call(
        _tc_mm_body,
        out_shape=jax.ShapeDtypeStruct((N, D_hid), jnp.float32),
    )(x, W1)

    dis16, y1 = pl.pallas_call(
        _tc1_body,
        out_shape=[
            jax.ShapeDtypeStruct((NP, D_hid), jnp.float32),
            jax.ShapeDtypeStruct((NP, D_hid), jnp.float32),
        ],
    )(deg_parts, xw)

    agg_kernel = _make_agg_kernel(NP, NCH, D_hid)

    agg1 = agg_kernel(y1, ei, zrows)

    y2 = pl.pallas_call(
        _tc2_body,
        out_shape=jax.ShapeDtypeStruct((NP, D_hid), jnp.float32),
    )(agg1, y1, dis16, b1.reshape(1, D_hid))

    agg2 = agg_kernel(y2, ei, zrows)

    out = pl.pallas_call(
        _tc3_body,
        out_shape=jax.ShapeDtypeStruct((N, D_out), jnp.float32),
    )(agg2, y2, dis16, W2, b2.reshape(1, D_out))

    return out


# speedup vs baseline: 69.0847x; 1.1507x over previous
import functools

import jax
import jax.numpy as jnp
from jax import lax
from jax.experimental import pallas as pl
from jax.experimental.pallas import tpu as pltpu
from jax.experimental.pallas import tpu_sc as plsc

_NC, _NS, _L = 2, 16, 16
_NW = _NC * _NS
_CH = 128
_CG = 16


def _make_deg_kernel(NP, NCH, NW):
    mesh = plsc.VectorSubcoreMesh(core_axis_name="c", subcore_axis_name="s",
                                  num_cores=_NC, num_subcores=_NS)

    @functools.partial(
        pl.kernel, mesh=mesh,
        out_type=jax.ShapeDtypeStruct((NW, NP), jnp.float32),
        scratch_types=[
            pltpu.VMEM((NCH, _CH), jnp.int32),
            pltpu.VMEM((NP,), jnp.float32),
        ],
        compiler_params=pltpu.CompilerParams(needs_layout_passes=False),
    )
    def deg_kernel(ei_hbm, zeros_hbm, out_hbm, idx_v, hist_v):
        wid = lax.axis_index("s") * _NC + lax.axis_index("c")
        pltpu.sync_copy(ei_hbm.at[1, wid], idx_v)
        pltpu.sync_copy(zeros_hbm, hist_v)
        ones = jnp.full((_L,), 1.0, jnp.float32)

        def body(i, carry):
            for u in range(_CH // _L):
                idx = idx_v[i, pl.ds(u * _L, _L)]
                plsc.addupdate_scatter(hist_v, [idx], ones)
            return carry

        lax.fori_loop(0, NCH, body, 0)
        pltpu.sync_copy(hist_v, out_hbm.at[wid])

    return deg_kernel


def _make_agg_kernel(NP, NCH, D):
    mesh = plsc.VectorSubcoreMesh(core_axis_name="c", subcore_axis_name="s",
                                  num_cores=_NC, num_subcores=_NS)
    RPT = NP // _NS
    FPT = RPT * D // 128

    NG = -(-NCH // _CG)

    @functools.partial(
        pl.kernel, mesh=mesh,
        out_type=jax.ShapeDtypeStruct((_NC, NP, D), jnp.float32),
        scratch_types=[
            pltpu.VMEM((NCH, _CH), jnp.int32),
            pltpu.VMEM((NCH, _CH), jnp.int32),
            pltpu.VMEM((2 * _CG * _CH, D), jnp.float32),
            pltpu.VMEM_SHARED((NP, D), jnp.float32),
            pltpu.SemaphoreType.DMA,
            pltpu.SemaphoreType.DMA,
            pltpu.SemaphoreType.DMA,
            pltpu.SemaphoreType.DMA,
        ],
        compiler_params=pltpu.CompilerParams(
            needs_layout_passes=False, use_tc_tiling_on_sc=False),
    )
    def agg_kernel(y_hbm, ei_hbm, zrows_hbm, out_hbm,
                   sidx_v, didx_v, rows_v, acc_sh,
                   gsem0, gsem1, ssem0, ssem1):
        c = lax.axis_index("c")
        s = lax.axis_index("s")
        wid = s * _NC + c
        gsems = (gsem0, gsem1)
        ssems = (ssem0, ssem1)
        with jax.named_scope("agg_stage"):
            pltpu.sync_copy(zrows_hbm.at[pl.ds(s * RPT, RPT)],
                            acc_sh.at[pl.ds(s * RPT, RPT)])
            pltpu.sync_copy(ei_hbm.at[0, wid], sidx_v)
            pltpu.sync_copy(ei_hbm.at[1, wid], didx_v)
            plsc.subcore_barrier()

        def chunks(g):
            return range(g * _CG, min((g + 1) * _CG, NCH))

        def issue_gathers(g):
            buf = (g % 2) * _CG * _CH
            return [
                pltpu.async_copy(
                    y_hbm.at[sidx_v.at[k]],
                    rows_v.at[pl.ds(buf + (k - g * _CG) * _CH, _CH)],
                    gsems[g % 2])
                for k in chunks(g)
            ]

        def issue_scatters(g):
            buf = (g % 2) * _CG * _CH
            return [
                pltpu.async_copy(
                    rows_v.at[pl.ds(buf + (k - g * _CG) * _CH, _CH)],
                    acc_sh.at[didx_v.at[k]],
                    ssems[g % 2], add=True)
                for k in chunks(g)
            ]

        with jax.named_scope("agg_edges"):
            gd = {0: issue_gathers(0)}
            sd = {}
            for g in range(NG):
                if g + 1 < NG:
                    for d in sd.pop(g - 1, ()):
                        d.wait()
                    gd[g + 1] = issue_gathers(g + 1)
                for d in gd.pop(g):
                    d.wait()
                sd[g] = issue_scatters(g)
            for g in sorted(sd):
                for d in sd[g]:
                    d.wait()

        with jax.named_scope("agg_dump"):
            plsc.subcore_barrier()
            pltpu.sync_copy(acc_sh.at[pl.ds(s * RPT, RPT)],
                            out_hbm.at[c, pl.ds(s * RPT, RPT), :])

    return agg_kernel


def _tc_mm_body(x_ref, w1_ref, xw_ref):
    xw_ref[...] = jnp.dot(x_ref[...], w1_ref[...],
                          preferred_element_type=jnp.float32)


def _tc1_body(parts_ref, xw_ref, dis_ref, y1_ref):
    N = xw_ref.shape[0]
    NP = dis_ref.shape[0]
    D = xw_ref.shape[1]
    deg = jnp.sum(parts_ref[...], axis=0) + 1.0
    dis = lax.rsqrt(deg)
    dis_ref[...] = jnp.broadcast_to(dis[:, None], (NP, D))
    y1_ref[...] = dis_ref[...] * jnp.pad(xw_ref[...], ((0, NP - N), (0, 0)))


def _tc2_body(agg_ref, y1_ref, dis_ref, b1_ref, y2_ref):
    a = agg_ref[0] + agg_ref[1] + y1_ref[...]
    h = jnp.maximum(dis_ref[...] * a + b1_ref[...], 0.0)
    y2_ref[...] = dis_ref[...] * h


def _tc3_body(agg_ref, y2_ref, dis_ref, w2_ref, b2_ref, out_ref):
    N = out_ref.shape[0]
    a2 = dis_ref[...] * (agg_ref[0] + agg_ref[1] + y2_ref[...])
    out_ref[...] = (
        jnp.dot(a2[:N, :], w2_ref[...], preferred_element_type=jnp.float32)
        + b2_ref[...]
    )


def kernel(x, edge_index, W1, b1, W2, b2):
    N, D_in = x.shape
    D_hid = W1.shape[1]
    D_out = W2.shape[1]
    E = edge_index.shape[1]

    NP = -(-(N + 1) // 128) * 128
    NCH = -(-E // (_NW * _CH))
    EPW = NCH * _CH
    E_pad = _NW * EPW
    F = NP * D_hid // 128
    NF = N * D_hid // 128
    FPT = F // _NS

    ei = edge_index.astype(jnp.int32)
    pad_idx = N + jnp.arange(E_pad - E, dtype=jnp.int32) % (NP - N)
    ei = jnp.concatenate(
        [ei, jnp.broadcast_to(pad_idx, (2, E_pad - E))], axis=1)
    ei = lax.optimization_barrier(ei.reshape(2, _NW, NCH, _CH))

    zeros_np = jnp.zeros((NP,), jnp.float32)
    zrows = jnp.zeros((NP, D_hid), jnp.float32)

    deg_parts = _make_deg_kernel(NP, NCH, _NW)(ei, zeros_np)

    xw = pl.pallas_call(
        _tc_mm_body,
        out_shape=jax.ShapeDtypeStruct((N, D_hid), jnp.float32),
    )(x, W1)

    dis16, y1 = pl.pallas_call(
        _tc1_body,
        out_shape=[
            jax.ShapeDtypeStruct((NP, D_hid), jnp.float32),
            jax.ShapeDtypeStruct((NP, D_hid), jnp.float32),
        ],
    )(deg_parts, xw)

    agg_kernel = _make_agg_kernel(NP, NCH, D_hid)

    agg1 = agg_kernel(y1, ei, zrows)

    y2 = pl.pallas_call(
        _tc2_body,
        out_shape=jax.ShapeDtypeStruct((NP, D_hid), jnp.float32),
    )(agg1, y1, dis16, b1.reshape(1, D_hid))

    agg2 = agg_kernel(y2, ei, zrows)

    out = pl.pallas_call(
        _tc3_body,
        out_shape=jax.ShapeDtypeStruct((N, D_out), jnp.float32),
    )(agg2, y2, dis16, W2, b2.reshape(1, D_out))

    return out
